# baseline (speedup 1.0000x reference)
import functools

import jax
import jax.numpy as jnp
from jax import lax
from jax.experimental import pallas as pl
from jax.experimental.pallas import tpu as pltpu
from jax.experimental.pallas import tpu_sc as plsc

NC = 2
NS = 16
NW = NC * NS
CHUNK = 128


NBUF = 2
GID = 8


def _sc_segment_sum(h2, src_idx, dst_idx, zeros, n_pad, k, d):
  rpz = n_pad // NS
  ng = k // GID
  assert k % GID == 0 and ng >= 2
  mesh = plsc.VectorSubcoreMesh(core_axis_name="c", subcore_axis_name="s")

  @functools.partial(
      pl.kernel,
      out_type=jax.ShapeDtypeStruct((NC, n_pad, d), jnp.float32),
      mesh=mesh,
      scratch_types=[
          pltpu.VMEM((k, CHUNK), jnp.int32),
          pltpu.VMEM((GID, CHUNK), jnp.int32),
          pltpu.VMEM((NBUF, CHUNK, d), jnp.float32),
          pltpu.VMEM_SHARED((n_pad, d), jnp.float32),
      ] + [pltpu.SemaphoreType.DMA] * NBUF,
  )
  def body(h2_hbm, src_hbm, dst_hbm, zero_hbm, out_hbm, src_v, dst_v, rows_v,
           agg_sh, *gsem):
    c = lax.axis_index("c")
    s = lax.axis_index("s")
    wid = c * NS + s
    pltpu.sync_copy(src_hbm.at[wid], src_v)
    pltpu.sync_copy(zero_hbm, agg_sh.at[pl.ds(s * rpz, rpz)])
    plsc.subcore_barrier()

    def start_g(b, j):
      pltpu.async_copy(h2_hbm.at[src_v.at[j]], rows_v.at[b], gsem[b])

    def wait_g(b, j):
      pltpu.make_async_copy(h2_hbm.at[src_v.at[j]], rows_v.at[b],
                            gsem[b]).wait()

    def scat(b, p):
      pltpu.sync_copy(rows_v.at[b], agg_sh.at[dst_v.at[p]], add=True)

    start_g(0, 0)
    start_g(1, 1)

    @pl.loop(0, ng - 1)
    def _(g):
      pltpu.sync_copy(dst_hbm.at[wid, pl.ds(g * GID, GID)], dst_v)
      for p in range(GID):
        j = g * GID + p
        b = p % NBUF
        wait_g(b, j)
        scat(b, p)
        start_g(b, j + NBUF)

    g0 = ng - 1
    pltpu.sync_copy(dst_hbm.at[wid, pl.ds(g0 * GID, GID)], dst_v)
    for p in range(GID):
      j = g0 * GID + p
      b = p % NBUF
      wait_g(b, j)
      scat(b, p)
      if j + NBUF < k:
        start_g(b, j + NBUF)

    plsc.subcore_barrier()
    pltpu.sync_copy(agg_sh.at[pl.ds(s * rpz, rpz)],
                    out_hbm.at[c, pl.ds(s * rpz, rpz)])

  return body(h2, src_idx, dst_idx, zeros)


def _sc_degree(dst_idx, ones, zeros, n_pad, k):
  rpz = n_pad // NS
  mesh = plsc.VectorSubcoreMesh(core_axis_name="c", subcore_axis_name="s")

  @functools.partial(
      pl.kernel,
      out_type=jax.ShapeDtypeStruct((NC, n_pad, 16), jnp.float32),
      mesh=mesh,
      scratch_types=[
          pltpu.VMEM((k, CHUNK), jnp.int32),
          pltpu.VMEM((CHUNK, 16), jnp.float32),
          pltpu.VMEM_SHARED((n_pad, 16), jnp.float32),
      ],
  )
  def body(dst_hbm, ones_hbm, zero_hbm, out_hbm, dst_v, ones_v, deg_sh):
    c = lax.axis_index("c")
    s = lax.axis_index("s")
    wid = c * NS + s
    pltpu.sync_copy(dst_hbm.at[wid], dst_v)
    pltpu.sync_copy(ones_hbm, ones_v)
    pltpu.sync_copy(zero_hbm, deg_sh.at[pl.ds(s * rpz, rpz)])
    plsc.subcore_barrier()

    @pl.loop(0, k)
    def _(j):
      pltpu.sync_copy(ones_v, deg_sh.at[dst_v.at[j]], add=True)

    plsc.subcore_barrier()
    pltpu.sync_copy(deg_sh.at[pl.ds(s * rpz, rpz)],
                    out_hbm.at[c, pl.ds(s * rpz, rpz)])

  return body(dst_idx, ones, zeros)


def _tc_matmul(x, w):
  def body(x_ref, w_ref, o_ref):
    o_ref[...] = jnp.dot(x_ref[...], w_ref[...],
                         preferred_element_type=jnp.float32)

  return pl.pallas_call(
      body,
      out_shape=jax.ShapeDtypeStruct((x.shape[0], w.shape[1]), jnp.float32),
  )(x, w)


def _tc_scale(p, deg, n):
  def body(p_ref, deg_ref, o_ref):
    dinv = lax.rsqrt(deg_ref[0, :n, 0:1] + deg_ref[1, :n, 0:1])
    o_ref[...] = dinv * p_ref[...]

  return pl.pallas_call(
      body,
      out_shape=jax.ShapeDtypeStruct(p.shape, jnp.float32),
  )(p, deg)


def _tc_mid(agg, deg, b, w, n):
  def body(agg_ref, deg_ref, b_ref, w_ref, o_ref):
    dinv = lax.rsqrt(deg_ref[0, :n, 0:1] + deg_ref[1, :n, 0:1])
    h = jnp.maximum(dinv * (agg_ref[0, :n, :] + agg_ref[1, :n, :])
                    + b_ref[...], 0.0)
    o_ref[...] = dinv * jnp.dot(h, w_ref[...],
                                preferred_element_type=jnp.float32)

  return pl.pallas_call(
      body,
      out_shape=jax.ShapeDtypeStruct((n, w.shape[1]), jnp.float32),
  )(agg, deg, b, w)


def _tc_final(agg, deg, b, n, d_out):
  def body(agg_ref, deg_ref, b_ref, o_ref):
    dinv = lax.rsqrt(deg_ref[0, :n, 0:1] + deg_ref[1, :n, 0:1])
    o_ref[...] = jax.nn.sigmoid(
        dinv * (agg_ref[0, :n, :d_out] + agg_ref[1, :n, :d_out]) + b_ref[...])

  return pl.pallas_call(
      body,
      out_shape=jax.ShapeDtypeStruct((n, d_out), jnp.float32),
  )(agg, deg, b)


def kernel(x, edge_index, W1, b1, W2, b2, W3, b3, W4, b4):
  n, d_in = x.shape
  e = edge_index.shape[1]
  d_hid = W1.shape[1]
  d_out = W4.shape[1]

  tot = e + n
  per = NW * CHUNK
  k = -(-tot // per)
  k = -(-k // 8) * 8
  e_pad = k * per
  n_pad = -(-n // (NS * CHUNK)) * (NS * CHUNK)

  loop = jnp.arange(n, dtype=jnp.int32)
  src = jnp.concatenate([edge_index[0].astype(jnp.int32), loop])
  dst = jnp.concatenate([edge_index[1].astype(jnp.int32), loop])
  npad_e = e_pad - tot
  pad_dst = n + (jnp.arange(npad_e, dtype=jnp.int32) % (n_pad - n))
  src = jnp.pad(src, (0, npad_e)).reshape(NW, k, CHUNK)
  dst = jnp.concatenate([dst, pad_dst]).reshape(NW, k, CHUNK)

  rpz = n_pad // NS
  zeros_hid = jnp.zeros((rpz, d_hid), jnp.float32)
  zeros_16 = jnp.zeros((rpz, 16), jnp.float32)
  ones_16 = jnp.ones((CHUNK, 16), jnp.float32)

  b1r = b1.reshape(1, -1)
  b2r = b2.reshape(1, -1)
  b3r = b3.reshape(1, -1)
  b4r = b4.reshape(1, -1)
  w4p = jnp.pad(W4, ((0, 0), (0, 128 - d_out)))

  deg = _sc_degree(dst, ones_16, zeros_16, n_pad, k)
  p1 = _tc_matmul(x, W1)
  h2 = _tc_scale(p1, deg, n)
  agg = _sc_segment_sum(h2, src, dst, zeros_hid, n_pad, k, d_hid)
  h2 = _tc_mid(agg, deg, b1r, W2, n)
  agg = _sc_segment_sum(h2, src, dst, zeros_hid, n_pad, k, d_hid)
  h2 = _tc_mid(agg, deg, b2r, W3, n)
  agg = _sc_segment_sum(h2, src, dst, zeros_hid, n_pad, k, d_hid)
  h2 = _tc_mid(agg, deg, b3r, w4p, n)
  agg = _sc_segment_sum(h2, src, dst, zeros_hid, n_pad, k, 128)
  return _tc_final(agg, deg, b4r, n, d_out)


# speedup vs baseline: 5.2914x; 5.2914x over previous
import functools

import jax
import jax.numpy as jnp
from jax import lax
from jax.experimental import pallas as pl
from jax.experimental.pallas import tpu as pltpu
from jax.experimental.pallas import tpu_sc as plsc

NC = 2
NS = 16
NW = NC * NS
CHUNK = 128


def _sc_segment_sum(h2, src_idx, dst_idx, zeros, n_pad, k, d):
  rpz = n_pad // NS
  mesh = plsc.VectorSubcoreMesh(core_axis_name="c", subcore_axis_name="s")

  @functools.partial(
      pl.kernel,
      out_type=jax.ShapeDtypeStruct((NC, n_pad, d), jnp.float32),
      mesh=mesh,
      scratch_types=[
          pltpu.VMEM((k, CHUNK), jnp.int32),
          pltpu.VMEM((k, CHUNK), jnp.int32),
          pltpu.VMEM((CHUNK, d), jnp.float32),
          pltpu.VMEM_SHARED((n_pad, d), jnp.float32),
      ],
  )
  def body(h2_hbm, src_hbm, dst_hbm, zero_hbm, out_hbm, src_v, dst_v, rows_v,
           agg_sh):
    c = lax.axis_index("c")
    s = lax.axis_index("s")
    wid = c * NS + s
    pltpu.sync_copy(src_hbm.at[wid], src_v)
    pltpu.sync_copy(dst_hbm.at[wid], dst_v)
    pltpu.sync_copy(zero_hbm, agg_sh.at[pl.ds(s * rpz, rpz)])
    plsc.subcore_barrier()

    @pl.loop(0, k)
    def _(j):
      pltpu.sync_copy(h2_hbm.at[src_v.at[j]], rows_v)
      pltpu.sync_copy(rows_v, agg_sh.at[dst_v.at[j]], add=True)

    plsc.subcore_barrier()
    pltpu.sync_copy(agg_sh.at[pl.ds(s * rpz, rpz)],
                    out_hbm.at[c, pl.ds(s * rpz, rpz)])

  return body(h2, src_idx, dst_idx, zeros)


def _sc_degree(dst_idx, ones, zeros, n_pad, k):
  rpz = n_pad // NS
  mesh = plsc.VectorSubcoreMesh(core_axis_name="c", subcore_axis_name="s")

  @functools.partial(
      pl.kernel,
      out_type=jax.ShapeDtypeStruct((NC, n_pad, 16), jnp.float32),
      mesh=mesh,
      scratch_types=[
          pltpu.VMEM((k, CHUNK), jnp.int32),
          pltpu.VMEM((CHUNK, 16), jnp.float32),
          pltpu.VMEM_SHARED((n_pad, 16), jnp.float32),
      ],
  )
  def body(dst_hbm, ones_hbm, zero_hbm, out_hbm, dst_v, ones_v, deg_sh):
    c = lax.axis_index("c")
    s = lax.axis_index("s")
    wid = c * NS + s
    pltpu.sync_copy(dst_hbm.at[wid], dst_v)
    pltpu.sync_copy(ones_hbm, ones_v)
    pltpu.sync_copy(zero_hbm, deg_sh.at[pl.ds(s * rpz, rpz)])
    plsc.subcore_barrier()

    @pl.loop(0, k)
    def _(j):
      pltpu.sync_copy(ones_v, deg_sh.at[dst_v.at[j]], add=True)

    plsc.subcore_barrier()
    pltpu.sync_copy(deg_sh.at[pl.ds(s * rpz, rpz)],
                    out_hbm.at[c, pl.ds(s * rpz, rpz)])

  return body(dst_idx, ones, zeros)


def _tc_matmul(x, w):
  def body(x_ref, w_ref, o_ref):
    o_ref[...] = jnp.dot(x_ref[...], w_ref[...],
                         preferred_element_type=jnp.float32)

  return pl.pallas_call(
      body,
      out_shape=jax.ShapeDtypeStruct((x.shape[0], w.shape[1]), jnp.float32),
  )(x, w)


def _tc_scale(p, deg, n):
  def body(p_ref, deg_ref, o_ref):
    dinv = lax.rsqrt(deg_ref[0, :n, 0:1] + deg_ref[1, :n, 0:1])
    o_ref[...] = dinv * p_ref[...]

  return pl.pallas_call(
      body,
      out_shape=jax.ShapeDtypeStruct(p.shape, jnp.float32),
  )(p, deg)


def _tc_mid(agg, deg, b, w, n):
  def body(agg_ref, deg_ref, b_ref, w_ref, o_ref):
    dinv = lax.rsqrt(deg_ref[0, :n, 0:1] + deg_ref[1, :n, 0:1])
    h = jnp.maximum(dinv * (agg_ref[0, :n, :] + agg_ref[1, :n, :])
                    + b_ref[...], 0.0)
    o_ref[...] = dinv * jnp.dot(h, w_ref[...],
                                preferred_element_type=jnp.float32)

  return pl.pallas_call(
      body,
      out_shape=jax.ShapeDtypeStruct((n, w.shape[1]), jnp.float32),
  )(agg, deg, b, w)


def _tc_relu_scale(agg, deg, b, n):
  d = agg.shape[-1]

  def body(agg_ref, deg_ref, b_ref, o_ref):
    dinv = lax.rsqrt(deg_ref[0, :n, 0:1] + deg_ref[1, :n, 0:1])
    o_ref[...] = dinv * jnp.maximum(
        dinv * (agg_ref[0, :n, :] + agg_ref[1, :n, :]) + b_ref[...], 0.0)

  return pl.pallas_call(
      body,
      out_shape=jax.ShapeDtypeStruct((n, d), jnp.float32),
  )(agg, deg, b)


def _tc_final(agg, deg, b, w, n, d_out):
  def body(agg_ref, deg_ref, b_ref, w_ref, o_ref):
    dinv = lax.rsqrt(deg_ref[0, :n, 0:1] + deg_ref[1, :n, 0:1])
    a = agg_ref[0, :n, :] + agg_ref[1, :n, :]
    o_ref[...] = jax.nn.sigmoid(
        dinv * jnp.dot(a, w_ref[...], preferred_element_type=jnp.float32)
        + b_ref[...])

  return pl.pallas_call(
      body,
      out_shape=jax.ShapeDtypeStruct((n, d_out), jnp.float32),
  )(agg, deg, b, w)


def kernel(x, edge_index, W1, b1, W2, b2, W3, b3, W4, b4):
  n, d_in = x.shape
  e = edge_index.shape[1]
  d_hid = W1.shape[1]
  d_out = W4.shape[1]

  tot = e + n
  per = NW * CHUNK
  k = -(-tot // per)
  e_pad = k * per
  n_pad = -(-n // (NS * CHUNK)) * (NS * CHUNK)

  loop = jnp.arange(n, dtype=jnp.int32)
  src = jnp.concatenate([edge_index[0].astype(jnp.int32), loop])
  dst = jnp.concatenate([edge_index[1].astype(jnp.int32), loop])
  npad_e = e_pad - tot
  pad_dst = n + (jnp.arange(npad_e, dtype=jnp.int32) % (n_pad - n))
  src = jnp.pad(src, (0, npad_e)).reshape(NW, k, CHUNK)
  dst = jnp.concatenate([dst, pad_dst]).reshape(NW, k, CHUNK)

  rpz = n_pad // NS
  zeros_hid = jnp.zeros((rpz, d_hid), jnp.float32)
  zeros_16 = jnp.zeros((rpz, 16), jnp.float32)
  ones_16 = jnp.ones((CHUNK, 16), jnp.float32)

  b1r = b1.reshape(1, -1)
  b2r = b2.reshape(1, -1)
  b3r = b3.reshape(1, -1)
  b4r = b4.reshape(1, -1)

  deg = _sc_degree(dst, ones_16, zeros_16, n_pad, k)
  p1 = _tc_matmul(x, W1)
  h2 = _tc_scale(p1, deg, n)
  agg = _sc_segment_sum(h2, src, dst, zeros_hid, n_pad, k, d_hid)
  h2 = _tc_mid(agg, deg, b1r, W2, n)
  agg = _sc_segment_sum(h2, src, dst, zeros_hid, n_pad, k, d_hid)
  h2 = _tc_mid(agg, deg, b2r, W3, n)
  agg = _sc_segment_sum(h2, src, dst, zeros_hid, n_pad, k, d_hid)
  h2 = _tc_relu_scale(agg, deg, b3r, n)
  agg = _sc_segment_sum(h2, src, dst, zeros_hid, n_pad, k, d_hid)
  return _tc_final(agg, deg, b4r, W4, n, d_out)


# speedup vs baseline: 10.5143x; 1.9870x over previous
import functools

import jax
import jax.numpy as jnp
from jax import lax
from jax.experimental import pallas as pl
from jax.experimental.pallas import tpu as pltpu
from jax.experimental.pallas import tpu_sc as plsc

NC = 2
NS = 16
NW = NC * NS
CHUNK = 128
NBUF = 2
GID = 8


def _sc_segment_sum(h2, src_idx, dst_idx, zeros, n_pad, k, d):
  rpz = n_pad // NS
  ng = k // GID
  assert k % GID == 0 and ng >= 2
  mesh = plsc.VectorSubcoreMesh(core_axis_name="c", subcore_axis_name="s")

  @functools.partial(
      pl.kernel,
      out_type=jax.ShapeDtypeStruct((NC, n_pad, d), jnp.float32),
      mesh=mesh,
      scratch_types=[
          pltpu.VMEM((k, CHUNK), jnp.int32),
          pltpu.VMEM((GID, CHUNK), jnp.int32),
          pltpu.VMEM((NBUF, CHUNK, d), jnp.float32),
          pltpu.VMEM_SHARED((n_pad, d), jnp.float32),
      ] + [pltpu.SemaphoreType.DMA] * NBUF,
  )
  def body(h2_hbm, src_hbm, dst_hbm, zero_hbm, out_hbm, src_v, dst_v, rows_v,
           agg_sh, *gsem):
    c = lax.axis_index("c")
    s = lax.axis_index("s")
    wid = c * NS + s
    pltpu.sync_copy(src_hbm.at[wid], src_v)
    pltpu.sync_copy(zero_hbm, agg_sh.at[pl.ds(s * rpz, rpz)])
    plsc.subcore_barrier()

    def start_g(b, j):
      pltpu.async_copy(h2_hbm.at[src_v.at[j]], rows_v.at[b], gsem[b])

    def wait_g(b, j):
      pltpu.make_async_copy(h2_hbm.at[src_v.at[j]], rows_v.at[b],
                            gsem[b]).wait()

    def scat(b, p):
      pltpu.sync_copy(rows_v.at[b], agg_sh.at[dst_v.at[p]], add=True)

    start_g(0, 0)
    start_g(1, 1)

    @pl.loop(0, ng - 1)
    def _(g):
      pltpu.sync_copy(dst_hbm.at[wid, pl.ds(g * GID, GID)], dst_v)
      for p in range(GID):
        j = g * GID + p
        b = p % NBUF
        wait_g(b, j)
        scat(b, p)
        start_g(b, j + NBUF)

    g0 = ng - 1
    pltpu.sync_copy(dst_hbm.at[wid, pl.ds(g0 * GID, GID)], dst_v)
    for p in range(GID):
      j = g0 * GID + p
      b = p % NBUF
      wait_g(b, j)
      scat(b, p)
      if j + NBUF < k:
        start_g(b, j + NBUF)

    plsc.subcore_barrier()
    pltpu.sync_copy(agg_sh.at[pl.ds(s * rpz, rpz)],
                    out_hbm.at[c, pl.ds(s * rpz, rpz)])

  return body(h2, src_idx, dst_idx, zeros)


def _sc_degree(dst_idx, ones, zeros, n_pad, k):
  rpz = n_pad // NS
  mesh = plsc.VectorSubcoreMesh(core_axis_name="c", subcore_axis_name="s")

  @functools.partial(
      pl.kernel,
      out_type=jax.ShapeDtypeStruct((NC, n_pad, 16), jnp.float32),
      mesh=mesh,
      scratch_types=[
          pltpu.VMEM((k, CHUNK), jnp.int32),
          pltpu.VMEM((CHUNK, 16), jnp.float32),
          pltpu.VMEM_SHARED((n_pad, 16), jnp.float32),
      ],
  )
  def body(dst_hbm, ones_hbm, zero_hbm, out_hbm, dst_v, ones_v, deg_sh):
    c = lax.axis_index("c")
    s = lax.axis_index("s")
    wid = c * NS + s
    pltpu.sync_copy(dst_hbm.at[wid], dst_v)
    pltpu.sync_copy(ones_hbm, ones_v)
    pltpu.sync_copy(zero_hbm, deg_sh.at[pl.ds(s * rpz, rpz)])
    plsc.subcore_barrier()

    @pl.loop(0, k)
    def _(j):
      pltpu.sync_copy(ones_v, deg_sh.at[dst_v.at[j]], add=True)

    plsc.subcore_barrier()
    pltpu.sync_copy(deg_sh.at[pl.ds(s * rpz, rpz)],
                    out_hbm.at[c, pl.ds(s * rpz, rpz)])

  return body(dst_idx, ones, zeros)


def _tc_matmul(x, w):
  def body(x_ref, w_ref, o_ref):
    o_ref[...] = jnp.dot(x_ref[...], w_ref[...],
                         preferred_element_type=jnp.float32)

  return pl.pallas_call(
      body,
      out_shape=jax.ShapeDtypeStruct((x.shape[0], w.shape[1]), jnp.float32),
  )(x, w)


def _tc_scale(p, deg, n):
  n_pad = deg.shape[1]
  d = p.shape[-1]

  def body(p_ref, deg_ref, o_ref):
    dinv = lax.rsqrt(deg_ref[0, :n, 0:1] + deg_ref[1, :n, 0:1])
    o_ref[:n, :] = dinv * p_ref[...]
    o_ref[n:, :] = jnp.zeros((n_pad - n, d), jnp.float32)

  return pl.pallas_call(
      body,
      out_shape=jax.ShapeDtypeStruct((n_pad, d), jnp.float32),
  )(p, deg)


def _tc_mid(agg, deg, b, w, n):
  n_pad = deg.shape[1]
  d = w.shape[1]

  def body(agg_ref, deg_ref, b_ref, w_ref, o_ref):
    dinv = lax.rsqrt(deg_ref[0, :n, 0:1] + deg_ref[1, :n, 0:1])
    h = jnp.maximum(dinv * (agg_ref[0, :n, :] + agg_ref[1, :n, :])
                    + b_ref[...], 0.0)
    o_ref[:n, :] = dinv * jnp.dot(h, w_ref[...],
                                  preferred_element_type=jnp.float32)
    o_ref[n:, :] = jnp.zeros((n_pad - n, d), jnp.float32)

  return pl.pallas_call(
      body,
      out_shape=jax.ShapeDtypeStruct((n_pad, d), jnp.float32),
  )(agg, deg, b, w)


def _tc_relu_scale(agg, deg, b, n):
  n_pad = deg.shape[1]
  d = agg.shape[-1]

  def body(agg_ref, deg_ref, b_ref, o_ref):
    dinv = lax.rsqrt(deg_ref[0, :n, 0:1] + deg_ref[1, :n, 0:1])
    o_ref[:n, :] = dinv * jnp.maximum(
        dinv * (agg_ref[0, :n, :] + agg_ref[1, :n, :]) + b_ref[...], 0.0)
    o_ref[n:, :] = jnp.zeros((n_pad - n, d), jnp.float32)

  return pl.pallas_call(
      body,
      out_shape=jax.ShapeDtypeStruct((n_pad, d), jnp.float32),
  )(agg, deg, b)


def _tc_final(agg, deg, b, w, n, d_out):
  def body(agg_ref, deg_ref, b_ref, w_ref, o_ref):
    dinv = lax.rsqrt(deg_ref[0, :n, 0:1] + deg_ref[1, :n, 0:1])
    a = agg_ref[0, :n, :] + agg_ref[1, :n, :]
    o_ref[...] = jax.nn.sigmoid(
        dinv * jnp.dot(a, w_ref[...], preferred_element_type=jnp.float32)
        + b_ref[...])

  return pl.pallas_call(
      body,
      out_shape=jax.ShapeDtypeStruct((n, d_out), jnp.float32),
  )(agg, deg, b, w)


def kernel(x, edge_index, W1, b1, W2, b2, W3, b3, W4, b4):
  n, d_in = x.shape
  e = edge_index.shape[1]
  d_hid = W1.shape[1]
  d_out = W4.shape[1]

  tot = e + n
  per = NW * CHUNK
  k = -(-tot // per)
  k = -(-k // GID) * GID
  e_pad = k * per
  n_pad = -(-n // (NS * CHUNK)) * (NS * CHUNK)

  loop = jnp.arange(n, dtype=jnp.int32)
  src = jnp.concatenate([edge_index[0].astype(jnp.int32), loop])
  dst = jnp.concatenate([edge_index[1].astype(jnp.int32), loop])
  npad_e = e_pad - tot
  pad_src = n + (jnp.arange(npad_e, dtype=jnp.int32) % (n_pad - n))
  pad_dst = jnp.arange(npad_e, dtype=jnp.int32) % n_pad
  pad_dst_deg = n + (jnp.arange(npad_e, dtype=jnp.int32) % (n_pad - n))
  src = jnp.concatenate([src, pad_src]).reshape(NW, k, CHUNK)
  dst_deg = jnp.concatenate([dst, pad_dst_deg]).reshape(NW, k, CHUNK)
  dst = jnp.concatenate([dst, pad_dst]).reshape(NW, k, CHUNK)

  rpz = n_pad // NS
  zeros_hid = jnp.zeros((rpz, d_hid), jnp.float32)
  zeros_16 = jnp.zeros((rpz, 16), jnp.float32)
  ones_16 = jnp.ones((CHUNK, 16), jnp.float32)

  b1r = b1.reshape(1, -1)
  b2r = b2.reshape(1, -1)
  b3r = b3.reshape(1, -1)
  b4r = b4.reshape(1, -1)

  deg = _sc_degree(dst_deg, ones_16, zeros_16, n_pad, k)
  p1 = _tc_matmul(x, W1)
  h2 = _tc_scale(p1, deg, n)
  agg = _sc_segment_sum(h2, src, dst, zeros_hid, n_pad, k, d_hid)
  h2 = _tc_mid(agg, deg, b1r, W2, n)
  agg = _sc_segment_sum(h2, src, dst, zeros_hid, n_pad, k, d_hid)
  h2 = _tc_mid(agg, deg, b2r, W3, n)
  agg = _sc_segment_sum(h2, src, dst, zeros_hid, n_pad, k, d_hid)
  h2 = _tc_relu_scale(agg, deg, b3r, n)
  agg = _sc_segment_sum(h2, src, dst, zeros_hid, n_pad, k, d_hid)
  return _tc_final(agg, deg, b4r, W4, n, d_out)
